# baseline (speedup 1.0000x reference)
import jax
import jax.numpy as jnp
from jax.experimental import pallas as pl
from jax.experimental.pallas import tpu as pltpu

T = 50
S = 16384
BS = 2048
GRID = S // BS


def _body(p_ref, g_ref, loss_ref, cnt_ref):
    step = pl.program_id(0)

    g = g_ref[...]
    p = p_ref[...]
    e = g - p
    s = e * e
    rs = s[:, 0, :] + s[:, 1, :]
    norm = jnp.sqrt(rs)
    m = g[:, 0, :] != 0.0
    part_l = jnp.sum(jnp.where(m, norm, 0.0))
    part_c = jnp.sum(jnp.where(m, 1.0, 0.0))

    @pl.when(step == 0)
    def _init():
        loss_ref[0, 0] = 0.0
        cnt_ref[0, 0] = 0.0

    loss_ref[0, 0] += part_l
    cnt_ref[0, 0] += part_c


@jax.jit
def kernel(pred_out, pred_gt):
    pt = jnp.transpose(pred_out, (1, 2, 0))
    gt = jnp.transpose(pred_gt, (1, 2, 0))
    loss, cnt = pl.pallas_call(
        _body,
        grid=(GRID,),
        in_specs=[
            pl.BlockSpec((T, 2, BS), lambda i: (0, 0, i)),
            pl.BlockSpec((T, 2, BS), lambda i: (0, 0, i)),
        ],
        out_specs=[
            pl.BlockSpec((1, 1), lambda i: (0, 0), memory_space=pltpu.SMEM),
            pl.BlockSpec((1, 1), lambda i: (0, 0), memory_space=pltpu.SMEM),
        ],
        out_shape=[
            jax.ShapeDtypeStruct((1, 1), jnp.float32),
            jax.ShapeDtypeStruct((1, 1), jnp.float32),
        ],
    )(pt, gt)
    return loss[0, 0], cnt[0, 0].astype(jnp.int32)


# speedup vs baseline: 1.0820x; 1.0820x over previous
import jax
import jax.numpy as jnp
from jax.experimental import pallas as pl
from jax.experimental.pallas import tpu as pltpu

T = 50
S = 16384
BS = 4096
GRID = S // BS


def _body(p_ref, g_ref, loss_ref, cnt_ref):
    step = pl.program_id(0)

    g = g_ref[...]
    p = p_ref[...]
    e = g - p
    s = e * e
    rs = s[:, 0, :] + s[:, 1, :]
    norm = jnp.sqrt(rs)
    m = g[:, 0, :] != 0.0
    part_l = jnp.sum(jnp.where(m, norm, 0.0))
    part_c = jnp.sum(jnp.where(m, 1.0, 0.0))

    @pl.when(step == 0)
    def _init():
        loss_ref[0, 0] = 0.0
        cnt_ref[0, 0] = 0.0

    loss_ref[0, 0] += part_l
    cnt_ref[0, 0] += part_c


@jax.jit
def kernel(pred_out, pred_gt):
    pt = jnp.transpose(pred_out, (1, 2, 0))
    gt = jnp.transpose(pred_gt, (1, 2, 0))
    loss, cnt = pl.pallas_call(
        _body,
        grid=(GRID,),
        in_specs=[
            pl.BlockSpec((T, 2, BS), lambda i: (0, 0, i)),
            pl.BlockSpec((T, 2, BS), lambda i: (0, 0, i)),
        ],
        out_specs=[
            pl.BlockSpec((1, 1), lambda i: (0, 0), memory_space=pltpu.SMEM),
            pl.BlockSpec((1, 1), lambda i: (0, 0), memory_space=pltpu.SMEM),
        ],
        out_shape=[
            jax.ShapeDtypeStruct((1, 1), jnp.float32),
            jax.ShapeDtypeStruct((1, 1), jnp.float32),
        ],
    )(pt, gt)
    return loss[0, 0], cnt[0, 0].astype(jnp.int32)
